# baseline (speedup 1.0000x reference)
import jax
import jax.numpy as jnp
from jax.experimental import pallas as pl
from jax.experimental.pallas import tpu as pltpu

D = 768
H = 768
E = 8
O = 3
N = 2048
TILE = 512


def _router_kernel(x_ref, pW1_ref, pb1_ref, pW2_ref, pb2_ref,
                   probs_ref, init_ref):
    x = x_ref[...]
    h = jnp.maximum(
        jnp.dot(x, pW1_ref[...], preferred_element_type=jnp.float32)
        + pb1_ref[...], 0.0)
    logits = jnp.dot(h, pW2_ref[...],
                     preferred_element_type=jnp.float32) + pb2_ref[...]
    m = jnp.max(logits, axis=-1, keepdims=True)
    ex = jnp.exp(logits - m)
    p = ex / jnp.sum(ex, axis=-1, keepdims=True)
    probs_ref[...] = p
    lane = jax.lax.broadcasted_iota(jnp.int32, (N, E + 1), 1)
    pid_col = jnp.sum(jnp.where(lane == E, p, 0.0), axis=1, keepdims=True)
    init_ref[...] = pid_col * x


def _expert_kernel(x_ref, probs_ref, init_ref,
                   projW_ref, projb_ref, rl_ref,
                   W1_ref, b1_ref, W2_ref, b2_ref,
                   out_ref):
    e = pl.program_id(1)
    xb = x_ref[...]

    xp = jnp.dot(xb, projW_ref[0], preferred_element_type=jnp.float32)
    xp = xp + projb_ref[0]

    rl = rl_ref[0]
    rm = jnp.max(rl, axis=-1, keepdims=True)
    re_ = jnp.exp(rl - rm)
    w = re_ / jnp.sum(re_, axis=-1, keepdims=True)

    b1 = b1_ref[0]
    b2 = b2_ref[0]
    acc = jnp.zeros((TILE, H), jnp.float32)
    for o in range(O):
        h1 = jnp.maximum(
            jnp.dot(xp, W1_ref[0, o], preferred_element_type=jnp.float32)
            + b1[o:o + 1], 0.0)
        h2 = jnp.maximum(
            jnp.dot(h1, W2_ref[0, o], preferred_element_type=jnp.float32)
            + b2[o:o + 1], 0.0)
        acc = acc + w[:, o:o + 1] * h2

    lane = jax.lax.broadcasted_iota(jnp.int32, (TILE, E + 1), 1)
    pcol = jnp.sum(jnp.where(lane == e, probs_ref[...], 0.0),
                   axis=1, keepdims=True)
    contrib = pcol * acc

    @pl.when(e == 0)
    def _init():
        out_ref[...] = init_ref[...] + contrib

    @pl.when(e != 0)
    def _accum():
        out_ref[...] += contrib


def kernel(x, proj_W, proj_b, route_logits, op_W1, op_b1, op_W2, op_b2,
           pW1, pb1, pW2, pb2):
    probs, init = pl.pallas_call(
        _router_kernel,
        out_shape=(
            jax.ShapeDtypeStruct((N, E + 1), jnp.float32),
            jax.ShapeDtypeStruct((N, D), jnp.float32),
        ),
    )(x, pW1, pb1.reshape(1, H), pW2, pb2.reshape(1, E + 1))

    grid = (N // TILE, E)
    out = pl.pallas_call(
        _expert_kernel,
        grid=grid,
        in_specs=[
            pl.BlockSpec((TILE, D), lambda t, e: (t, 0)),
            pl.BlockSpec((TILE, E + 1), lambda t, e: (t, 0)),
            pl.BlockSpec((TILE, D), lambda t, e: (t, 0)),
            pl.BlockSpec((1, D, H), lambda t, e: (e, 0, 0)),
            pl.BlockSpec((1, 1, H), lambda t, e: (e, 0, 0)),
            pl.BlockSpec((1, 1, O), lambda t, e: (e, 0, 0)),
            pl.BlockSpec((1, O, H, H), lambda t, e: (e, 0, 0, 0)),
            pl.BlockSpec((1, O, H), lambda t, e: (e, 0, 0)),
            pl.BlockSpec((1, O, H, H), lambda t, e: (e, 0, 0, 0)),
            pl.BlockSpec((1, O, H), lambda t, e: (e, 0, 0)),
        ],
        out_specs=pl.BlockSpec((TILE, H), lambda t, e: (t, 0)),
        out_shape=jax.ShapeDtypeStruct((N, H), jnp.float32),
        compiler_params=pltpu.CompilerParams(
            dimension_semantics=("arbitrary", "arbitrary"),
        ),
    )(
        x,
        probs,
        init,
        proj_W,
        proj_b.reshape(E, 1, H),
        route_logits.reshape(E, 1, O),
        op_W1,
        op_b1,
        op_W2,
        op_b2,
    )
    return out


# speedup vs baseline: 1.1230x; 1.1230x over previous
import jax
import jax.numpy as jnp
from jax.experimental import pallas as pl
from jax.experimental.pallas import tpu as pltpu

D = 768
H = 768
E = 8
O = 3
N = 2048


def _router_kernel(x_ref, pW1_ref, pb1_ref, pW2_ref, pb2_ref, probs_ref):
    x = x_ref[...]
    h = jnp.maximum(
        jnp.dot(x, pW1_ref[...], preferred_element_type=jnp.float32)
        + pb1_ref[...], 0.0)
    logits = jnp.dot(h, pW2_ref[...],
                     preferred_element_type=jnp.float32) + pb2_ref[...]
    m = jnp.max(logits, axis=-1, keepdims=True)
    ex = jnp.exp(logits - m)
    probs_ref[...] = ex / jnp.sum(ex, axis=-1, keepdims=True)


def _expert_kernel(x_ref, probs_ref,
                   projW_ref, projb_ref, rl_ref,
                   W1_ref, b1_ref, W2_ref, b2_ref,
                   out_ref):
    e = pl.program_id(0)
    xb = x_ref[...]

    xp = jnp.dot(xb, projW_ref[0], preferred_element_type=jnp.float32)
    xp = xp + projb_ref[0]

    rl = rl_ref[0]
    rm = jnp.max(rl, axis=-1, keepdims=True)
    re_ = jnp.exp(rl - rm)
    w = re_ / jnp.sum(re_, axis=-1, keepdims=True)

    pr = probs_ref[...]
    lane = jax.lax.broadcasted_iota(jnp.int32, (N, E + 1), 1)
    pcol = jnp.sum(jnp.where(lane == e, pr, 0.0), axis=1, keepdims=True)

    @pl.when(e == 0)
    def _init():
        pid_col = jnp.sum(jnp.where(lane == E, pr, 0.0),
                          axis=1, keepdims=True)
        out_ref[...] = pid_col * xb

    b1 = b1_ref[0]
    b2 = b2_ref[0]
    for o in range(O):
        h1 = jnp.maximum(
            jnp.dot(xp, W1_ref[0, o], preferred_element_type=jnp.float32)
            + b1[o:o + 1], 0.0)
        h2 = jnp.maximum(
            jnp.dot(h1, W2_ref[0, o], preferred_element_type=jnp.float32)
            + b2[o:o + 1], 0.0)
        out_ref[...] += (pcol * w[:, o:o + 1]) * h2


def kernel(x, proj_W, proj_b, route_logits, op_W1, op_b1, op_W2, op_b2,
           pW1, pb1, pW2, pb2):
    probs = pl.pallas_call(
        _router_kernel,
        out_shape=jax.ShapeDtypeStruct((N, E + 1), jnp.float32),
    )(x, pW1, pb1.reshape(1, H), pW2, pb2.reshape(1, E + 1))

    out = pl.pallas_call(
        _expert_kernel,
        grid=(E,),
        in_specs=[
            pl.BlockSpec((N, D), lambda e: (0, 0)),
            pl.BlockSpec((N, E + 1), lambda e: (0, 0)),
            pl.BlockSpec((1, D, H), lambda e: (e, 0, 0)),
            pl.BlockSpec((1, 1, H), lambda e: (e, 0, 0)),
            pl.BlockSpec((1, 1, O), lambda e: (e, 0, 0)),
            pl.BlockSpec((1, O, H, H), lambda e: (e, 0, 0, 0)),
            pl.BlockSpec((1, O, H), lambda e: (e, 0, 0)),
            pl.BlockSpec((1, O, H, H), lambda e: (e, 0, 0, 0)),
            pl.BlockSpec((1, O, H), lambda e: (e, 0, 0)),
        ],
        out_specs=pl.BlockSpec((N, H), lambda e: (0, 0)),
        out_shape=jax.ShapeDtypeStruct((N, H), jnp.float32),
        compiler_params=pltpu.CompilerParams(
            dimension_semantics=("arbitrary",),
            vmem_limit_bytes=63 * 1024 * 1024,
        ),
    )(
        x,
        probs,
        proj_W,
        proj_b.reshape(E, 1, H),
        route_logits.reshape(E, 1, O),
        op_W1,
        op_b1,
        op_W2,
        op_b2,
    )
    return out
